# baseline (speedup 1.0000x reference)
import functools

import jax
import jax.numpy as jnp
from jax import lax
from jax.experimental import pallas as pl
from jax.experimental.pallas import tpu as pltpu
from jax.experimental.pallas import tpu_sc as plsc

VOCAB = 100000
DIM = 64
BATCH = 1024

_NC = 2
_NS = 16
_NW = _NC * _NS
_BPW = BATCH // _NW
_DIM2 = 2 * DIM

_TM = 512
_SPLIT = 98 * _TM


def _pack_body(lo_ref, hi_ref, out_ref):
    gcol = _SPLIT + pl.program_id(0) * _TM + lax.broadcasted_iota(
        jnp.int32, (DIM, _TM), 1
    )
    hi = jnp.where(gcol < VOCAB, hi_ref[...], 0.0)
    stack = jnp.concatenate([lo_ref[...], hi], axis=0)
    ident = jnp.eye(_DIM2, dtype=jnp.float32)
    out_ref[...] = lax.dot_general(
        stack,
        ident,
        (((0,), (0,)), ((), ())),
        preferred_element_type=jnp.float32,
    )


def _pack_pairs(wt):
    grid = _SPLIT // _TM
    return pl.pallas_call(
        _pack_body,
        grid=(grid,),
        in_specs=[
            pl.BlockSpec((DIM, _TM), lambda i: (0, i)),
            pl.BlockSpec((DIM, _TM), lambda i: (0, i + _SPLIT // _TM)),
        ],
        out_specs=pl.BlockSpec((_TM, _DIM2), lambda i: (i, 0)),
        out_shape=jax.ShapeDtypeStruct((_SPLIT, _DIM2), jnp.float32),
        compiler_params=pltpu.CompilerParams(
            dimension_semantics=("parallel",),
        ),
    )(wt, wt)


def _gather_body(table_hbm, idx_hbm, par_hbm, out_hbm,
                 idx_v, par_v, rows_v, z_v, sem):
    wid = lax.axis_index("s") * _NC + lax.axis_index("c")
    base = wid * _BPW
    pltpu.sync_copy(idx_hbm.at[pl.ds(base, _BPW)], idx_v)
    pltpu.sync_copy(par_hbm.at[pl.ds(base, _BPW)], par_v)
    pltpu.async_copy(table_hbm.at[idx_v], rows_v, sem).wait()

    lane = lax.iota(jnp.int32, 16)

    def _one_row(i, carry):
        row = jnp.full((16,), i, jnp.int32)
        pv = plsc.load_gather(par_v, [row])
        for c in range(DIM // 16):
            col = DIM * pv + c * 16 + lane
            val = plsc.load_gather(rows_v, [row, col])
            plsc.store_scatter(z_v, [row, c * 16 + lane], val)
        return carry

    lax.fori_loop(0, _BPW, _one_row, 0)
    pltpu.sync_copy(z_v, out_hbm.at[pl.ds(base, _BPW)])


def _sc_gather(packed, idx, par):
    call = functools.partial(
        pl.kernel,
        mesh=plsc.VectorSubcoreMesh(core_axis_name="c", subcore_axis_name="s"),
        out_type=jax.ShapeDtypeStruct((BATCH, DIM), jnp.float32),
        scratch_types=[
            pltpu.VMEM((_BPW,), jnp.int32),
            pltpu.VMEM((_BPW,), jnp.int32),
            pltpu.VMEM((_BPW, _DIM2), jnp.float32),
            pltpu.VMEM((_BPW, DIM), jnp.float32),
            pltpu.SemaphoreType.DMA,
        ],
        compiler_params=pltpu.CompilerParams(needs_layout_passes=False),
    )(_gather_body)
    return call(packed, idx, par)


_TV = 2048


def _matmul_body(z_ref, wt_ref, b_ref, out_ref):
    out_ref[...] = (
        lax.dot_general(
            wt_ref[...],
            z_ref[...],
            (((0,), (1,)), ((), ())),
            preferred_element_type=jnp.float32,
        )
        + b_ref[...].reshape(_TV, 1)
    )


def _projection(z, out_Wt, out_b):
    grid = pl.cdiv(VOCAB, _TV)
    return pl.pallas_call(
        _matmul_body,
        grid=(grid,),
        in_specs=[
            pl.BlockSpec((BATCH, DIM), lambda i: (0, 0)),
            pl.BlockSpec((DIM, _TV), lambda i: (0, i)),
            pl.BlockSpec((_TV,), lambda i: (i,)),
        ],
        out_specs=pl.BlockSpec((_TV, BATCH), lambda i: (i, 0)),
        out_shape=jax.ShapeDtypeStruct((VOCAB, BATCH), jnp.float32),
        compiler_params=pltpu.CompilerParams(
            dimension_semantics=("parallel",),
        ),
    )(z, out_Wt, out_b)


def kernel(x, emb_table, out_W, out_b):
    xi = x.astype(jnp.int32)
    packed = _pack_pairs(emb_table.T)
    m = jnp.where(xi < _SPLIT, xi, xi - _SPLIT)
    p = (xi >= _SPLIT).astype(jnp.int32)
    z = _sc_gather(packed, m, p)
    logits_t = _projection(z, out_W.T, out_b)
    return logits_t.T


# speedup vs baseline: 1.2606x; 1.2606x over previous
import functools

import jax
import jax.numpy as jnp
from jax import lax
from jax.experimental import pallas as pl
from jax.experimental.pallas import tpu as pltpu
from jax.experimental.pallas import tpu_sc as plsc

VOCAB = 100000
DIM = 64
BATCH = 1024

_NC = 2
_NS = 16
_NW = _NC * _NS
_BPW = BATCH // _NW
_DIM2 = 2 * DIM

_TM = 4096
_SPLIT = 13 * _TM


def _pack_body(lo_ref, hi_ref, out_ref):
    gcol = _SPLIT + pl.program_id(0) * _TM + lax.broadcasted_iota(
        jnp.int32, (DIM, _TM), 1
    )
    hi = jnp.where(gcol < VOCAB, hi_ref[...], 0.0)
    stack = jnp.concatenate([lo_ref[...], hi], axis=0)
    ident = jnp.eye(_DIM2, dtype=jnp.float32)
    out_ref[...] = lax.dot_general(
        stack,
        ident,
        (((0,), (0,)), ((), ())),
        preferred_element_type=jnp.float32,
    )


def _pack_pairs(wt):
    grid = _SPLIT // _TM
    return pl.pallas_call(
        _pack_body,
        grid=(grid,),
        in_specs=[
            pl.BlockSpec((DIM, _TM), lambda i: (0, i)),
            pl.BlockSpec(
                (DIM, _TM),
                lambda i: (0, jnp.minimum(i + _SPLIT // _TM, (VOCAB - 1) // _TM)),
            ),
        ],
        out_specs=pl.BlockSpec((_TM, _DIM2), lambda i: (i, 0)),
        out_shape=jax.ShapeDtypeStruct((_SPLIT, _DIM2), jnp.float32),
        compiler_params=pltpu.CompilerParams(
            dimension_semantics=("parallel",),
        ),
    )(wt, wt)


def _gather_body(table_hbm, idx_hbm, par_hbm, out_hbm,
                 idx_v, par_v, rows_v, z_v, sem):
    wid = lax.axis_index("s") * _NC + lax.axis_index("c")
    base = wid * _BPW
    pltpu.sync_copy(idx_hbm.at[pl.ds(base, _BPW)], idx_v)
    pltpu.sync_copy(par_hbm.at[pl.ds(base, _BPW)], par_v)
    pltpu.async_copy(table_hbm.at[idx_v], rows_v, sem).wait()

    lane = lax.iota(jnp.int32, 16)

    def _one_row(i, carry):
        row = jnp.full((16,), i, jnp.int32)
        pv = plsc.load_gather(par_v, [row])
        for c in range(DIM // 16):
            col = DIM * pv + c * 16 + lane
            val = plsc.load_gather(rows_v, [row, col])
            plsc.store_scatter(z_v, [row, c * 16 + lane], val)
        return carry

    lax.fori_loop(0, _BPW, _one_row, 0)
    pltpu.sync_copy(z_v, out_hbm.at[pl.ds(base, _BPW)])


def _sc_gather(packed, idx, par):
    call = functools.partial(
        pl.kernel,
        mesh=plsc.VectorSubcoreMesh(core_axis_name="c", subcore_axis_name="s"),
        out_type=jax.ShapeDtypeStruct((BATCH, DIM), jnp.float32),
        scratch_types=[
            pltpu.VMEM((_BPW,), jnp.int32),
            pltpu.VMEM((_BPW,), jnp.int32),
            pltpu.VMEM((_BPW, _DIM2), jnp.float32),
            pltpu.VMEM((_BPW, DIM), jnp.float32),
            pltpu.SemaphoreType.DMA,
        ],
        compiler_params=pltpu.CompilerParams(needs_layout_passes=False),
    )(_gather_body)
    return call(packed, idx, par)


_TV = 2048


def _matmul_body(z_ref, wt_ref, b_ref, out_ref):
    out_ref[...] = (
        lax.dot_general(
            wt_ref[...],
            z_ref[...],
            (((0,), (1,)), ((), ())),
            preferred_element_type=jnp.float32,
        )
        + b_ref[...].reshape(_TV, 1)
    )


def _projection(z, out_Wt, out_b):
    grid = pl.cdiv(VOCAB, _TV)
    return pl.pallas_call(
        _matmul_body,
        grid=(grid,),
        in_specs=[
            pl.BlockSpec((BATCH, DIM), lambda i: (0, 0)),
            pl.BlockSpec((DIM, _TV), lambda i: (0, i)),
            pl.BlockSpec((_TV,), lambda i: (i,)),
        ],
        out_specs=pl.BlockSpec((_TV, BATCH), lambda i: (i, 0)),
        out_shape=jax.ShapeDtypeStruct((VOCAB, BATCH), jnp.float32),
        compiler_params=pltpu.CompilerParams(
            dimension_semantics=("parallel",),
        ),
    )(z, out_Wt, out_b)


def kernel(x, emb_table, out_W, out_b):
    xi = x.astype(jnp.int32)
    packed = _pack_pairs(emb_table.T)
    m = jnp.where(xi < _SPLIT, xi, xi - _SPLIT)
    p = (xi >= _SPLIT).astype(jnp.int32)
    z = _sc_gather(packed, m, p)
    logits_t = _projection(z, out_W.T, out_b)
    return logits_t.T


# speedup vs baseline: 1.2844x; 1.0188x over previous
import functools

import jax
import jax.numpy as jnp
from jax import lax
from jax.experimental import pallas as pl
from jax.experimental.pallas import tpu as pltpu
from jax.experimental.pallas import tpu_sc as plsc

VOCAB = 100000
DIM = 64
BATCH = 1024

_NC = 2
_NS = 16
_NW = _NC * _NS
_BPW = BATCH // _NW
_DIM2 = 2 * DIM

_TM = 4096
_SPLIT = 13 * _TM


def _pack_body(lo_ref, hi_ref, out_ref):
    gcol = _SPLIT + pl.program_id(0) * _TM + lax.broadcasted_iota(
        jnp.int32, (DIM, _TM), 1
    )
    hi = jnp.where(gcol < VOCAB, hi_ref[...], 0.0)
    stack = jnp.concatenate([lo_ref[...], hi], axis=0)
    ident = jnp.eye(_DIM2, dtype=jnp.float32)
    out_ref[...] = lax.dot_general(
        stack,
        ident,
        (((0,), (0,)), ((), ())),
        preferred_element_type=jnp.float32,
    )


def _pack_pairs(wt):
    grid = _SPLIT // _TM
    return pl.pallas_call(
        _pack_body,
        grid=(grid,),
        in_specs=[
            pl.BlockSpec((DIM, _TM), lambda i: (0, i)),
            pl.BlockSpec(
                (DIM, _TM),
                lambda i: (0, jnp.minimum(i + _SPLIT // _TM, (VOCAB - 1) // _TM)),
            ),
        ],
        out_specs=pl.BlockSpec((_TM, _DIM2), lambda i: (i, 0)),
        out_shape=jax.ShapeDtypeStruct((_SPLIT, _DIM2), jnp.float32),
        compiler_params=pltpu.CompilerParams(
            dimension_semantics=("parallel",),
        ),
    )(wt, wt)


def _gather_body(table_hbm, idx_hbm, par_hbm, out_hbm,
                 idx_v, par_v, rows_v, z_v, sem):
    wid = lax.axis_index("s") * _NC + lax.axis_index("c")
    base = wid * _BPW
    pltpu.sync_copy(idx_hbm.at[pl.ds(base, _BPW)], idx_v)
    pltpu.sync_copy(par_hbm.at[pl.ds(base, _BPW)], par_v)
    pltpu.async_copy(table_hbm.at[idx_v], rows_v, sem).wait()

    lane = lax.iota(jnp.int32, 16)

    def _one_row(i, carry):
        row = jnp.full((16,), i, jnp.int32)
        pv = plsc.load_gather(par_v, [row])
        for c in range(DIM // 16):
            col = DIM * pv + c * 16 + lane
            val = plsc.load_gather(rows_v, [row, col])
            plsc.store_scatter(z_v, [row, c * 16 + lane], val)
        return carry

    lax.fori_loop(0, _BPW, _one_row, 0)
    pltpu.sync_copy(z_v, out_hbm.at[pl.ds(base, _BPW)])


def _sc_gather(packed, idx, par):
    call = functools.partial(
        pl.kernel,
        mesh=plsc.VectorSubcoreMesh(core_axis_name="c", subcore_axis_name="s"),
        out_type=jax.ShapeDtypeStruct((BATCH, DIM), jnp.float32),
        scratch_types=[
            pltpu.VMEM((_BPW,), jnp.int32),
            pltpu.VMEM((_BPW,), jnp.int32),
            pltpu.VMEM((_BPW, _DIM2), jnp.float32),
            pltpu.VMEM((_BPW, DIM), jnp.float32),
            pltpu.SemaphoreType.DMA,
        ],
        compiler_params=pltpu.CompilerParams(needs_layout_passes=False),
    )(_gather_body)
    return call(packed, idx, par)


_TV = 4096


def _matmul_body(z_ref, wt_ref, b_ref, out_ref):
    out_ref[...] = (
        lax.dot_general(
            wt_ref[...],
            z_ref[...],
            (((0,), (1,)), ((), ())),
            preferred_element_type=jnp.float32,
        )
        + b_ref[...].reshape(_TV, 1)
    )


def _projection(z, out_Wt, out_b):
    grid = pl.cdiv(VOCAB, _TV)
    return pl.pallas_call(
        _matmul_body,
        grid=(grid,),
        in_specs=[
            pl.BlockSpec((BATCH, DIM), lambda i: (0, 0)),
            pl.BlockSpec((DIM, _TV), lambda i: (0, i)),
            pl.BlockSpec((_TV,), lambda i: (i,)),
        ],
        out_specs=pl.BlockSpec((_TV, BATCH), lambda i: (i, 0)),
        out_shape=jax.ShapeDtypeStruct((VOCAB, BATCH), jnp.float32),
        compiler_params=pltpu.CompilerParams(
            dimension_semantics=("parallel",),
        ),
    )(z, out_Wt, out_b)


def kernel(x, emb_table, out_W, out_b):
    xi = x.astype(jnp.int32)
    packed = _pack_pairs(emb_table.T)
    m = jnp.where(xi < _SPLIT, xi, xi - _SPLIT)
    p = (xi >= _SPLIT).astype(jnp.int32)
    z = _sc_gather(packed, m, p)
    logits_t = _projection(z, out_W.T, out_b)
    return logits_t.T
